# Initial kernel scaffold; baseline (speedup 1.0000x reference)
import jax
import jax.numpy as jnp
from jax.experimental import pallas as pl


def kernel(x, edge_attr, params, edge_index, line_edge_index, line_node_idx):
    raise NotImplementedError("write your pallas kernel here")


# speedup vs baseline: 2.9397x; 2.9397x over previous
import functools
import numpy as np

import jax
import jax.numpy as jnp
from jax import lax
from jax.experimental import pallas as pl
from jax.experimental.pallas import tpu as pltpu
from jax.experimental.pallas import tpu_sc as plsc

N_NODES = 10000
N_EDGES = 160000
N_LINE = 320000

_SC_CORES = 2
_SC_SUBCORES = 16
_NW = _SC_CORES * _SC_SUBCORES


def _divisor_at_most(m, cap, step=8):
    d = max(step, (cap // step) * step)
    while d >= step:
        if m % d == 0:
            return d
        d -= step
    return step


@functools.lru_cache(maxsize=None)
def _sc_gather_call(V, D, B):
    assert B % (_NW * 8) == 0, (V, D, B)
    per_w = B // _NW
    chunk = _divisor_at_most(per_w, max(8, (192 * 1024) // (4 * D)))
    n_it = per_w // chunk
    mesh = plsc.VectorSubcoreMesh(core_axis_name="c", subcore_axis_name="s")

    @functools.partial(
        pl.kernel, mesh=mesh,
        out_type=jax.ShapeDtypeStruct((B, D), jnp.float32),
        scratch_types=[
            pltpu.VMEM((chunk,), jnp.int32),
            pltpu.VMEM((chunk, D), jnp.float32),
            pltpu.SemaphoreType.DMA,
        ],
    )
    def k(tab_hbm, idx_hbm, out_hbm, idx_v, rows_v, sem):
        wid = lax.axis_index("s") * _SC_CORES + lax.axis_index("c")
        base = wid * per_w

        def body(it, carry):
            off = base + it * chunk
            pltpu.sync_copy(idx_hbm.at[pl.ds(off, chunk)], idx_v)
            pltpu.async_copy(tab_hbm.at[idx_v], rows_v, sem).wait()
            pltpu.sync_copy(rows_v, out_hbm.at[pl.ds(off, chunk)])
            return carry

        lax.fori_loop(0, n_it, body, 0, unroll=False)

    return k


def _gather(tab, idx):
    V, D = tab.shape
    B = idx.shape[0]
    if D % 128 != 0:
        return tab[idx]
    return _sc_gather_call(V, D, B)(tab, idx)


@functools.lru_cache(maxsize=None)
def _sc_segsum_node_call(E, D, n):
    NP = ((n + 2559) // 2560) * 2560
    rpt = NP // _SC_SUBCORES
    assert rpt % 8 == 0
    ept = E // _NW
    assert E % (_NW * 8) == 0
    bs = _divisor_at_most(ept, max(8, (128 * 1024) // (4 * max(D, 128))))
    mesh = plsc.VectorSubcoreMesh(core_axis_name="c", subcore_axis_name="s")

    @functools.partial(
        pl.kernel, mesh=mesh,
        out_type=(jax.ShapeDtypeStruct((n, D), jnp.float32),
                  jax.ShapeDtypeStruct((n, D), jnp.float32)),
        scratch_types=[
            pltpu.VMEM_SHARED((NP, D), jnp.float32),
            pltpu.VMEM((bs,), jnp.int32),
            pltpu.VMEM((bs, D), jnp.float32),
        ],
    )
    def k(vals_hbm, seg_hbm, out0, out1, acc, ibuf, vbuf):
        cid = lax.axis_index("c")
        sid = lax.axis_index("s")

        def zv(i, c):
            r = i // (D // 16)
            vbuf[r, pl.ds((i % (D // 16)) * 16, 16)] = jnp.zeros(
                (16,), jnp.float32)
            return c

        lax.fori_loop(0, bs * D // 16, zv, 0, unroll=False)
        zb = _divisor_at_most(rpt, bs)

        def zero_body(i, c):
            pltpu.sync_copy(vbuf.at[pl.ds(0, zb)],
                            acc.at[pl.ds(sid * rpt + i * zb, zb)])
            return c

        lax.fori_loop(0, rpt // zb, zero_body, 0, unroll=False)
        plsc.subcore_barrier()

        def body(i, c):
            e0 = (cid * _SC_SUBCORES + sid) * ept + i * bs
            pltpu.sync_copy(seg_hbm.at[pl.ds(e0, bs)], ibuf)
            pltpu.sync_copy(vals_hbm.at[pl.ds(e0, bs)], vbuf)
            pltpu.sync_copy(vbuf, acc.at[ibuf], add=True)
            return c

        lax.fori_loop(0, ept // bs, body, 0, unroll=False)
        plsc.subcore_barrier()

        out = [out0, out1]
        full = n // rpt
        rem = n - full * rpt
        for c in range(_SC_CORES):
            @pl.when((cid == c) & (sid < full))
            def _(c=c):
                pltpu.sync_copy(acc.at[pl.ds(sid * rpt, rpt)],
                                out[c].at[pl.ds(sid * rpt, rpt)])
            if rem:
                @pl.when((cid == c) & (sid == full))
                def _(c=c):
                    pltpu.sync_copy(acc.at[pl.ds(full * rpt, rem)],
                                    out[c].at[pl.ds(full * rpt, rem)])

    return k


def _segsum(vals, seg, n):
    E, D = vals.shape
    if True or D < 32 or n > 12800 or D > 128:
        return jax.ops.segment_sum(vals, seg, num_segments=n)
    p0, p1 = _sc_segsum_node_call(E, D, n)(vals, seg)
    return p0 + p1


_BS_BIN = 128
_FC = 80
_NPBIN = 10240
_DW = 128


@functools.lru_cache(maxsize=None)
def _sc_segsum_binned_call(E, D, n):
    bs = _BS_BIN
    NP = _NPBIN
    n_bins = (n + NP - 1) // NP
    assert n_bins <= 16 and n % _FC == 0 and D <= _DW
    rpt = NP // _SC_SUBCORES
    zb = _divisor_at_most(rpt, bs, 16)
    mesh = plsc.VectorSubcoreMesh(core_axis_name="c", subcore_axis_name="s")

    @functools.partial(
        pl.kernel, mesh=mesh,
        out_type=jax.ShapeDtypeStruct((n, D), jnp.float32),
        scratch_types=[
            pltpu.VMEM_SHARED((NP + 16, _DW), jnp.float32),
            pltpu.VMEM((32,), jnp.int32),
            pltpu.VMEM((bs,), jnp.int32),
            pltpu.VMEM((bs,), jnp.int32),
            pltpu.VMEM((bs, _DW), jnp.float32),
            pltpu.SemaphoreType.DMA,
        ],
    )
    def k(vals_hbm, sid_hbm, perm_hbm, bnd_hbm, out, acc, bbuf, ibuf,
          pbuf, vbuf, sem):
        cid = lax.axis_index("c")
        sid = lax.axis_index("s")
        pltpu.sync_copy(bnd_hbm, bbuf)
        v0 = bbuf[pl.ds(0, 16)]
        v1 = bbuf[pl.ds(16, 16)]
        lanes = lax.iota(jnp.int32, 16)

        def bsel(b):
            return (jnp.sum(v0 * (lanes == b).astype(jnp.int32)) +
                    jnp.sum(v1 * (lanes == b - 16).astype(jnp.int32)))

        def zv(i, c):
            r = i // (_DW // 16)
            col = (i % (_DW // 16)) * 16
            vbuf[r, pl.ds(col, 16)] = jnp.zeros((16,), jnp.float32)
            return c

        lax.fori_loop(0, bs * _DW // 16, zv, 0, unroll=False)

        for jj in range((n_bins + 1) // 2):
            b = cid + 2 * jj

            @pl.when(b < n_bins)
            def _(b=b):
                lo = b * NP
                nb_rows = jnp.minimum(n - lo, NP)
                s = (bsel(b) // 8) * 8
                e = bsel(b + 1)

                def zero_body(i, c):
                    pltpu.sync_copy(vbuf.at[pl.ds(0, zb)],
                                    acc.at[pl.ds(1 + sid * rpt + i * zb,
                                                 zb)])
                    return c

                lax.fori_loop(0, rpt // zb, zero_body, 0, unroll=False)
                plsc.subcore_barrier()

                nblk = (e - s + bs - 1) // bs
                mine = (nblk - sid + _SC_SUBCORES - 1) // _SC_SUBCORES

                def body(i, c):
                    off = s + (sid + i * _SC_SUBCORES) * bs
                    pltpu.sync_copy(sid_hbm.at[pl.ds(off, bs)], ibuf)
                    pltpu.sync_copy(perm_hbm.at[pl.ds(off, bs)], pbuf)
                    dst_v = vbuf if D == _DW else vbuf.at[:, pl.ds(0, D)]
                    pltpu.async_copy(vals_hbm.at[pbuf], dst_v, sem).wait()

                    def loc(j, c2):
                        v = ibuf[pl.ds(j * 16, 16)]
                        v = jnp.minimum(jnp.maximum(v - lo, -1), NP) + 1
                        ibuf[pl.ds(j * 16, 16)] = v
                        return c2

                    lax.fori_loop(0, bs // 16, loc, 0, unroll=False)
                    pltpu.sync_copy(vbuf, acc.at[ibuf], add=True)
                    return c

                lax.fori_loop(0, mine, body, 0, unroll=False)
                plsc.subcore_barrier()

                cnum = nb_rows // _FC
                myc = (cnum - sid + _SC_SUBCORES - 1) // _SC_SUBCORES
                src_a = acc if D == _DW else acc.at[:, pl.ds(0, D)]

                def flush(i, c):
                    r0 = (sid + i * _SC_SUBCORES) * _FC
                    pltpu.sync_copy(src_a.at[pl.ds(1 + r0, _FC)],
                                    out.at[pl.ds(lo + r0, _FC)])
                    return c

                lax.fori_loop(0, myc, flush, 0, unroll=False)
                plsc.subcore_barrier()

    return k


def _make_sortpack(seg, n):
    E = seg.shape[0]
    if n <= _NPBIN:
        sids = seg
        perm = jnp.arange(E, dtype=jnp.int32)
        bounds = jnp.concatenate(
            [jnp.zeros((1,), jnp.int32), jnp.full((16,), E, jnp.int32)])
    else:
        perm = jnp.argsort(seg).astype(jnp.int32)
        sids = jnp.take(seg, perm)
        edges = jnp.arange(0, 17, dtype=jnp.int32) * _NPBIN
        bounds = jnp.searchsorted(
            sids, jnp.minimum(edges, n), side="left").astype(jnp.int32)
    sids_p = jnp.concatenate([sids, jnp.full((_BS_BIN,), n, jnp.int32)])
    perm_p = jnp.concatenate([perm, jnp.zeros((_BS_BIN,), jnp.int32)])
    return {"seg": seg, "sids": sids_p, "perm": perm_p,
            "bounds": jnp.pad(bounds, (0, 32 - 17)), "n": n}


def _segsum2(vals, pack):
    E0, D = vals.shape
    return _sc_segsum_binned_call(E0, D, pack["n"])(
        vals, pack["sids"], pack["perm"], pack["bounds"])


_BM = 1024


def _rows_grid(M, bm=_BM):
    return (M + bm - 1) // bm


@functools.lru_cache(maxsize=None)
def _mm_call(M, K, N, act, have_add, slope):
    def body(*refs):
        if have_add:
            x_ref, w_ref, b_ref, a_ref, o_ref = refs
        else:
            x_ref, w_ref, b_ref, o_ref = refs
        y = jnp.dot(x_ref[...], w_ref[...],
                    preferred_element_type=jnp.float32) + b_ref[...]
        if have_add:
            y = y + a_ref[...]
        if act == "relu":
            y = jnp.maximum(y, 0.0)
        elif act == "leaky":
            y = jnp.where(y >= 0, y, np.float32(slope) * y)
        o_ref[...] = y

    in_specs = [
        pl.BlockSpec((_BM, K), lambda i: (i, 0)),
        pl.BlockSpec((K, N), lambda i: (0, 0)),
        pl.BlockSpec((1, N), lambda i: (0, 0)),
    ]
    if have_add:
        in_specs.append(pl.BlockSpec((_BM, N), lambda i: (i, 0)))
    return pl.pallas_call(
        body,
        grid=(_rows_grid(M),),
        in_specs=in_specs,
        out_specs=pl.BlockSpec((_BM, N), lambda i: (i, 0)),
        out_shape=jax.ShapeDtypeStruct((M, N), jnp.float32),
    )


def _mm(x, w, b=None, act=None, add=None, slope=0.01):
    M, K = x.shape
    N = w.shape[1]
    if b is None:
        b = jnp.zeros((N,), jnp.float32)
    args = (x, w, b.reshape(1, N))
    if add is not None:
        args = args + (add,)
    return _mm_call(M, K, N, act, add is not None, slope)(*args)


@functools.lru_cache(maxsize=None)
def _gine_tail_call(M, K, H, N):
    def body(x_ref, a_ref, e_ref, w1_ref, b1_ref, w2_ref, b2_ref, o_ref):
        h = x_ref[...] * e_ref[0, 0] + a_ref[...]
        t = jnp.dot(h, w1_ref[...], preferred_element_type=jnp.float32)
        t = t + b1_ref[...]
        t = jnp.where(t >= 0, t, np.float32(0.01) * t)
        y = jnp.dot(t, w2_ref[...], preferred_element_type=jnp.float32)
        o_ref[...] = y + b2_ref[...]

    return pl.pallas_call(
        body,
        grid=(_rows_grid(M),),
        in_specs=[
            pl.BlockSpec((_BM, K), lambda i: (i, 0)),
            pl.BlockSpec((_BM, K), lambda i: (i, 0)),
            pl.BlockSpec(memory_space=pltpu.SMEM),
            pl.BlockSpec((K, H), lambda i: (0, 0)),
            pl.BlockSpec((1, H), lambda i: (0, 0)),
            pl.BlockSpec((H, N), lambda i: (0, 0)),
            pl.BlockSpec((1, N), lambda i: (0, 0)),
        ],
        out_specs=pl.BlockSpec((_BM, N), lambda i: (i, 0)),
        out_shape=jax.ShapeDtypeStruct((M, N), jnp.float32),
    )


def _gine_tail(x, agg, eps, p1, p2):
    M, K = x.shape
    H = p1["w"].shape[1]
    N = p2["w"].shape[1]
    eps1 = (1.0 + eps).reshape(1, 1)
    return _gine_tail_call(M, K, H, N)(
        x, agg, eps1, p1["w"], p1["b"].reshape(1, H),
        p2["w"], p2["b"].reshape(1, N))


@functools.lru_cache(maxsize=None)
def _tfc_edge_call(E, D, heads):
    dh = D // heads
    scale = np.float32(1.0 / np.sqrt(dh))

    def body(q_ref, kv_ref, e_ref, al_ref, vv_ref):
        ee = e_ref[...]
        kk = kv_ref[:, :D] + ee
        vv_ref[...] = kv_ref[:, D:] + ee
        prod = q_ref[...] * kk
        al_ref[...] = prod.reshape(_BM, heads, dh).sum(-1) * scale

    return pl.pallas_call(
        body,
        grid=(_rows_grid(E),),
        in_specs=[
            pl.BlockSpec((_BM, D), lambda i: (i, 0)),
            pl.BlockSpec((_BM, 2 * D), lambda i: (i, 0)),
            pl.BlockSpec((_BM, D), lambda i: (i, 0)),
        ],
        out_specs=[
            pl.BlockSpec((_BM, heads), lambda i: (i, 0)),
            pl.BlockSpec((_BM, D), lambda i: (i, 0)),
        ],
        out_shape=[
            jax.ShapeDtypeStruct((E, heads), jnp.float32),
            jax.ShapeDtypeStruct((E, D), jnp.float32),
        ],
    )


@functools.lru_cache(maxsize=None)
def _attn_apply_call(E, D, heads):
    dh = D // heads

    def body(v_ref, w_ref, d_ref, o_ref):
        a = w_ref[...] / (d_ref[...] + np.float32(1e-16))
        a = jnp.repeat(a, dh, axis=1)
        o_ref[...] = v_ref[...] * a

    return pl.pallas_call(
        body,
        grid=(_rows_grid(E),),
        in_specs=[
            pl.BlockSpec((_BM, D), lambda i: (i, 0)),
            pl.BlockSpec((_BM, heads), lambda i: (i, 0)),
            pl.BlockSpec((_BM, heads), lambda i: (i, 0)),
        ],
        out_specs=pl.BlockSpec((_BM, D), lambda i: (i, 0)),
        out_shape=jax.ShapeDtypeStruct((E, D), jnp.float32),
    )


@functools.lru_cache(maxsize=None)
def _gat_score_call(E, D):
    def body(l_ref, r_ref, e_ref, att_ref, o_ref):
        m = l_ref[...] + r_ref[...] + e_ref[...]
        m = jnp.where(m >= 0, m, np.float32(0.2) * m)
        o_ref[...] = (m * att_ref[...]).sum(-1, keepdims=True)

    return pl.pallas_call(
        body,
        grid=(_rows_grid(E),),
        in_specs=[
            pl.BlockSpec((_BM, D), lambda i: (i, 0)),
            pl.BlockSpec((_BM, D), lambda i: (i, 0)),
            pl.BlockSpec((_BM, D), lambda i: (i, 0)),
            pl.BlockSpec((1, D), lambda i: (0, 0)),
        ],
        out_specs=pl.BlockSpec((_BM, 1), lambda i: (i, 0)),
        out_shape=jax.ShapeDtypeStruct((E, 1), jnp.float32),
    )


_CLAMP = 80.0


def _seg_softmax(alpha, seg, n, inv_deg, rounds):
    m = _segsum(alpha, seg, n) * inv_deg
    for r in range(rounds):
        md = _gather(m, seg)
        w = jnp.exp(jnp.minimum(alpha - md, _CLAMP))
        den = _segsum(w, seg, n)
        if r < rounds - 1:
            m = m + jnp.log(den)
    dend = _gather(den, seg)
    return w, dend


def _gine(p, x, src, dst, ea, n):
    xs = _gather(x, src)
    msg = _mm(ea, p["edge"]["w"], p["edge"]["b"], act="relu", add=xs)
    agg = _segsum(msg, dst, n)
    return _gine_tail(x, agg, p["eps"], p["mlp1"], p["mlp2"])


def _tfc(p, x, src, dst, ea, n, inv_deg, heads, dh, rounds):
    E = src.shape[0]
    D = heads * dh
    q = _mm(x, p["q"]["w"], p["q"]["b"])
    kv = _mm(x, jnp.concatenate([p["k"]["w"], p["v"]["w"]], axis=1),
             jnp.concatenate([p["k"]["b"], p["v"]["b"]]))
    ee = _mm(ea, p["e"]["w"], p["e"]["b"])
    qg = _gather(q, dst)
    kvg = _gather(kv, src)
    alpha, vv = _tfc_edge_call(E, D, heads)(qg, kvg, ee)
    w, dend = _seg_softmax(alpha, dst, n, inv_deg, rounds)
    msgv = _attn_apply_call(E, D, heads)(vv, w, dend)
    if D > 128:
        out = jnp.concatenate(
            [_segsum(msgv[:, :128], dst, n), _segsum(msgv[:, 128:], dst, n)],
            axis=1)
    else:
        out = _segsum(msgv, dst, n)
    return out + _mm(x, p["skip"]["w"], p["skip"]["b"])


def _gat(p, x, src, dst, ea, n, inv_deg, rounds):
    xl = _mm(x, p["l"]["w"], p["l"]["b"])
    xr = _mm(x, p["r"]["w"], p["r"]["b"])
    el = _mm(ea, p["e"]["w"], p["e"]["b"])
    xls = _gather(xl, src)
    xrd = _gather(xr, dst)
    E = src.shape[0]
    dout = xl.shape[1]
    score = _gat_score_call(E, dout)(xls, xrd, el, p["att"].reshape(1, dout))
    w, dend = _seg_softmax(score, dst, n, inv_deg, rounds)
    msgv = _attn_apply_call(E, dout, 1)(xls, w, dend)
    return _segsum(msgv, dst, n) + p["bias"]


def _seq(p, x, src, dst, ea, n, inv_deg, n_out, rounds=2):
    x = _gine(p["c1"], x, src, dst, ea, n)
    x = _tfc(p["c2"], x, src, dst, ea, n, inv_deg, 4, n_out, rounds)
    x = _tfc(p["c3"], x, src, dst, ea, n, inv_deg, 4, n_out, rounds)
    return _gat(p["c4"], x, src, dst, ea, n, inv_deg, rounds)


def _local(p, x, src, dst, ea, n, inv_deg, n_out, rounds=3):
    x = _gine(p["c1"], x, src, dst, ea, n)
    x = _gine(p["c2"], x, src, dst, ea, n)
    x = _tfc(p["c3"], x, src, dst, ea, n, inv_deg, 4, n_out, rounds)
    return _gat(p["c4"], x, src, dst, ea, n, inv_deg, rounds)


def kernel(x, edge_attr, params, edge_index, line_edge_index, line_node_idx):
    src, dst = edge_index[0], edge_index[1]
    lsrc, ldst = line_edge_index[0], line_edge_index[1]

    deg_n = _segsum(jnp.ones((N_EDGES, 1), jnp.float32), dst, N_NODES)
    inv_deg_n = 1.0 / jnp.maximum(deg_n, 1.0)
    deg_l = _segsum(jnp.ones((N_LINE, 1), jnp.float32), ldst, N_EDGES)
    inv_deg_l = 1.0 / jnp.maximum(deg_l, 1.0)

    x1 = _local(params["orbital"], x, src, dst, edge_attr, N_NODES,
                inv_deg_n, 64)
    line_ea = _gather(x1, line_node_idx)
    e1 = _seq(params["i_seq1"], edge_attr, lsrc, ldst, line_ea, N_EDGES,
              inv_deg_l, 32)
    e2 = _seq(params["i_seq2"], edge_attr, lsrc, ldst, line_ea, N_EDGES,
              inv_deg_l, 32)
    e3 = _seq(params["i_seq3"], edge_attr, lsrc, ldst, line_ea, N_EDGES,
              inv_deg_l, 32)
    w = params["i_weights"]
    ea2 = (e1 * w[0] + e2 * w[1] + e3 * w[2]) * np.float32(1.0 / 3.0)
    sbar = (w[0] + w[1] + w[2]) * np.float32(1.0 / 3.0)
    x2 = x1 * sbar
    onsite = _local(params["onsite"], x2, src, dst, ea2, N_NODES,
                    inv_deg_n, 32)
    line_x2 = line_ea * sbar
    hopping = _seq(params["hopping"], ea2, lsrc, ldst, line_x2, N_EDGES,
                   inv_deg_l, 32)
    return (onsite, hopping)
